# baseline (speedup 1.0000x reference)
import jax
import jax.numpy as jnp
from jax import lax
from jax.experimental import pallas as pl
from jax.experimental.pallas import tpu as pltpu
from jax.experimental.pallas import tpu_sc as plsc

B, C, H, W = 4, 96, 384, 384
HW = H * W
P = B * HW
NC, NS = 2, 16
NW = NC * NS
CHUNK = 96
CHUNKS_PER_WORKER = P // CHUNK // NW
WORKERS_PER_BATCH = NW // B
LG = 16
NPAIR = CHUNKS_PER_WORKER // 2
OROW = CHUNK * C // 128


def _warp_body(tbl, fx, fy, out, scratch):
    wid = lax.axis_index("s") * NC + lax.axis_index("c")
    bi = wid // WORKERS_PER_BATCH
    cb0 = (wid % WORKERS_PER_BATCH) * CHUNKS_PER_WORKER
    iota = lax.iota(jnp.int32, LG)

    def fire_flow(i, bk):
        p0 = bi * HW + (cb0 + i) * CHUNK
        pltpu.async_copy(fx.at[pl.ds(p0, CHUNK)], bk["fxb"], bk["fs"])
        pltpu.async_copy(fy.at[pl.ds(p0, CHUNK)], bk["fyb"], bk["fs"])

    def start(i, bk):
        pltpu.make_async_copy(fx.at[pl.ds(0, CHUNK)], bk["fxb"], bk["fs"]).wait()
        pltpu.make_async_copy(fy.at[pl.ds(0, CHUNK)], bk["fyb"], bk["fs"]).wait()
        cidx = cb0 + i
        y0 = cidx >> 2
        x0 = (cidx & 3) * CHUNK
        y0f = y0.astype(jnp.float32)
        for j in range(CHUNK // LG):
            sl = pl.ds(j * LG, LG)
            xb = (x0 + j * LG + iota).astype(jnp.float32)
            x2 = xb + bk["fxb"][sl]
            y2 = y0f + bk["fyb"][sl]
            ix_l = jnp.clip(x2.astype(jnp.int32), 0, W - 1)
            iy_t = jnp.clip(y2.astype(jnp.int32), 0, H - 1)
            ix_r = jnp.minimum(ix_l + 1, W - 1)
            iy_b = jnp.minimum(iy_t + 1, H - 1)
            a = x2 - ix_l.astype(jnp.float32)
            bt = y2 - iy_t.astype(jnp.float32)
            base_t = iy_t * W + (bi * HW)
            base_b = iy_b * W + (bi * HW)
            bk["idx"][0][sl] = base_t + ix_l
            bk["idx"][1][sl] = base_t + ix_r
            bk["idx"][2][sl] = base_b + ix_l
            bk["idx"][3][sl] = base_b + ix_r
            ra = 1.0 - a
            rb = 1.0 - bt
            bk["w"][0][sl] = ra * rb
            bk["w"][1][sl] = a * rb
            bk["w"][2][sl] = ra * bt
            bk["w"][3][sl] = a * bt
        for n in range(4):
            pltpu.async_copy(tbl.at[bk["idx"][n]], bk["rows"][n], bk["gs"])

    def finish(i, bk):
        for n in range(4):
            pltpu.make_async_copy(tbl.at[pl.ds(0, CHUNK), :], bk["rows"][n],
                                  bk["gs"]).wait()
        @pl.when(i >= 2)
        def _():
            pltpu.make_async_copy(out.at[pl.ds(0, CHUNK * C)], bk["out_t"],
                                  bk["os"]).wait()

        r0, r1, r2, r3 = bk["rows"]

        def kg_body(kg, carry2):
            ksl = pl.ds(kg * LG, LG)
            wtlv = bk["w"][0][ksl]
            wtrv = bk["w"][1][ksl]
            wblv = bk["w"][2][ksl]
            wbrv = bk["w"][3][ksl]
            for l in range(LG):
                k = kg * LG + l
                wtl = wtlv[l]
                wtr = wtrv[l]
                wbl = wblv[l]
                wbr = wbrv[l]
                for cg in range(C // LG):
                    sl = pl.ds(cg * LG, LG)
                    acc = (wtl * r0[k, sl] + wtr * r1[k, sl]
                           + wbl * r2[k, sl] + wbr * r3[k, sl])
                    bk["out_t"][pl.ds(k * C + cg * LG, LG)] = acc
            return carry2

        lax.fori_loop(0, CHUNK // LG, kg_body, 0)
        p0 = bi * HW + (cb0 + i) * CHUNK
        pltpu.async_copy(bk["out_t"], out.at[pl.ds(p0 * C, CHUNK * C)], bk["os"])

    ba, bb = scratch["a"], scratch["b"]

    fire_flow(0, ba)
    start(0, ba)
    fire_flow(1, bb)

    def pair_body(g, carry):
        not_last = g < NPAIR - 1

        @pl.when(not_last)
        def _():
            fire_flow(2 * g + 2, ba)

        start(2 * g + 1, bb)

        @pl.when(not_last)
        def _():
            fire_flow(2 * g + 3, bb)

        finish(2 * g, ba)

        @pl.when(not_last)
        def _():
            start(2 * g + 2, ba)

        finish(2 * g + 1, bb)
        return carry

    lax.fori_loop(0, NPAIR, pair_body, 0)
    pltpu.make_async_copy(out.at[pl.ds(0, CHUNK * C)], ba["out_t"], ba["os"]).wait()
    pltpu.make_async_copy(out.at[pl.ds(0, CHUNK * C)], bb["out_t"], bb["os"]).wait()


def _make_bank():
    return dict(
        idx=[pltpu.VMEM((CHUNK,), jnp.int32) for _ in range(4)],
        w=[pltpu.VMEM((CHUNK,), jnp.float32) for _ in range(4)],
        rows=[pltpu.VMEM((CHUNK, C), jnp.float32) for _ in range(4)],
        fxb=pltpu.VMEM((CHUNK,), jnp.float32),
        fyb=pltpu.VMEM((CHUNK,), jnp.float32),
        out_t=pltpu.VMEM((CHUNK * C,), jnp.float32),
        fs=pltpu.SemaphoreType.DMA,
        gs=pltpu.SemaphoreType.DMA,
        os=pltpu.SemaphoreType.DMA,
    )


@jax.jit
def _warp(tbl, fx, fy):
    mesh = plsc.VectorSubcoreMesh(core_axis_name="c", subcore_axis_name="s",
                                  num_cores=NC, num_subcores=NS)
    return pl.kernel(
        _warp_body,
        out_type=jax.ShapeDtypeStruct((P * C,), jnp.float32),
        mesh=mesh,
        compiler_params=pltpu.CompilerParams(use_tc_tiling_on_sc=False),
        scratch_types=[{"a": _make_bank(), "b": _make_bank()}],
    )(tbl, fx, fy)


def kernel(input1, input2):
    tbl = input1.transpose(0, 2, 3, 1).reshape(P, C)
    fx = input2[:, 0].reshape(P)
    fy = input2[:, 1].reshape(P)
    out = _warp(tbl, fx, fy)
    return out.reshape(B, H, W, C).transpose(0, 3, 1, 2)


# speedup vs baseline: 1.0082x; 1.0082x over previous
import jax
import jax.numpy as jnp
from jax import lax
from jax.experimental import pallas as pl
from jax.experimental.pallas import tpu as pltpu
from jax.experimental.pallas import tpu_sc as plsc

B, C, H, W = 4, 96, 384, 384
HW = H * W
NC, NS = 2, 16
NW = NC * NS
CHUNK = 96
CHUNKS_PER_WORKER = HW // CHUNK // NW
LG = 16
NPAIR = CHUNKS_PER_WORKER // 2


def _warp_body(tbl, fx, fy, out, scratch):
    wid = lax.axis_index("s") * NC + lax.axis_index("c")
    cb0 = wid * CHUNKS_PER_WORKER
    iota = lax.iota(jnp.int32, LG)

    def fire_flow(i, bk):
        p0 = (cb0 + i) * CHUNK
        pltpu.async_copy(fx.at[pl.ds(p0, CHUNK)], bk["fxb"], bk["fs"])
        pltpu.async_copy(fy.at[pl.ds(p0, CHUNK)], bk["fyb"], bk["fs"])

    def start(i, bk):
        pltpu.make_async_copy(fx.at[pl.ds(0, CHUNK)], bk["fxb"], bk["fs"]).wait()
        pltpu.make_async_copy(fy.at[pl.ds(0, CHUNK)], bk["fyb"], bk["fs"]).wait()
        cidx = cb0 + i
        y0 = cidx >> 2
        x0 = (cidx & 3) * CHUNK
        y0f = y0.astype(jnp.float32)
        for j in range(CHUNK // LG):
            sl = pl.ds(j * LG, LG)
            xb = (x0 + j * LG + iota).astype(jnp.float32)
            x2 = xb + bk["fxb"][sl]
            y2 = y0f + bk["fyb"][sl]
            ix_l = jnp.clip(x2.astype(jnp.int32), 0, W - 1)
            iy_t = jnp.clip(y2.astype(jnp.int32), 0, H - 1)
            ix_r = jnp.minimum(ix_l + 1, W - 1)
            iy_b = jnp.minimum(iy_t + 1, H - 1)
            a = x2 - ix_l.astype(jnp.float32)
            bt = y2 - iy_t.astype(jnp.float32)
            base_t = iy_t * W
            base_b = iy_b * W
            bk["idx"][0][sl] = base_t + ix_l
            bk["idx"][1][sl] = base_t + ix_r
            bk["idx"][2][sl] = base_b + ix_l
            bk["idx"][3][sl] = base_b + ix_r
            ra = 1.0 - a
            rb = 1.0 - bt
            bk["w"][0][sl] = ra * rb
            bk["w"][1][sl] = a * rb
            bk["w"][2][sl] = ra * bt
            bk["w"][3][sl] = a * bt
        for n in range(4):
            pltpu.async_copy(tbl.at[bk["idx"][n]], bk["rows"][n], bk["gs"])

    def finish(i, bk):
        for n in range(4):
            pltpu.make_async_copy(tbl.at[pl.ds(0, CHUNK), :], bk["rows"][n],
                                  bk["gs"]).wait()
        @pl.when(i >= 2)
        def _():
            pltpu.make_async_copy(tbl.at[pl.ds(0, CHUNK), :], bk["out_t"],
                                  bk["os"]).wait()

        r0, r1, r2, r3 = bk["rows"]

        def kg_body(kg, carry2):
            ksl = pl.ds(kg * LG, LG)
            wtlv = bk["w"][0][ksl]
            wtrv = bk["w"][1][ksl]
            wblv = bk["w"][2][ksl]
            wbrv = bk["w"][3][ksl]
            for l in range(LG):
                k = kg * LG + l
                wtl = wtlv[l]
                wtr = wtrv[l]
                wbl = wblv[l]
                wbr = wbrv[l]
                for cg in range(C // LG):
                    sl = pl.ds(cg * LG, LG)
                    acc = (wtl * r0[k, sl] + wtr * r1[k, sl]
                           + wbl * r2[k, sl] + wbr * r3[k, sl])
                    bk["out_t"][k, sl] = acc
            return carry2

        lax.fori_loop(0, CHUNK // LG, kg_body, 0)
        p0 = (cb0 + i) * CHUNK
        pltpu.async_copy(bk["out_t"], out.at[pl.ds(p0, CHUNK), :], bk["os"])

    ba, bb = scratch["a"], scratch["b"]

    fire_flow(0, ba)
    start(0, ba)
    fire_flow(1, bb)

    def pair_body(g, carry):
        not_last = g < NPAIR - 1

        @pl.when(not_last)
        def _():
            fire_flow(2 * g + 2, ba)

        start(2 * g + 1, bb)

        @pl.when(not_last)
        def _():
            fire_flow(2 * g + 3, bb)

        finish(2 * g, ba)

        @pl.when(not_last)
        def _():
            start(2 * g + 2, ba)

        finish(2 * g + 1, bb)
        return carry

    lax.fori_loop(0, NPAIR, pair_body, 0)
    pltpu.make_async_copy(tbl.at[pl.ds(0, CHUNK), :], ba["out_t"], ba["os"]).wait()
    pltpu.make_async_copy(tbl.at[pl.ds(0, CHUNK), :], bb["out_t"], bb["os"]).wait()


def _make_bank():
    return dict(
        idx=[pltpu.VMEM((CHUNK,), jnp.int32) for _ in range(4)],
        w=[pltpu.VMEM((CHUNK,), jnp.float32) for _ in range(4)],
        rows=[pltpu.VMEM((CHUNK, C), jnp.float32) for _ in range(4)],
        fxb=pltpu.VMEM((CHUNK,), jnp.float32),
        fyb=pltpu.VMEM((CHUNK,), jnp.float32),
        out_t=pltpu.VMEM((CHUNK, C), jnp.float32),
        fs=pltpu.SemaphoreType.DMA,
        gs=pltpu.SemaphoreType.DMA,
        os=pltpu.SemaphoreType.DMA,
    )


def _warp_image(tbl, fx, fy):
    mesh = plsc.VectorSubcoreMesh(core_axis_name="c", subcore_axis_name="s",
                                  num_cores=NC, num_subcores=NS)
    return pl.kernel(
        _warp_body,
        out_type=jax.ShapeDtypeStruct((HW, C), jnp.float32),
        mesh=mesh,
        compiler_params=pltpu.CompilerParams(use_tc_tiling_on_sc=False),
        scratch_types=[{"a": _make_bank(), "b": _make_bank()}],
    )(tbl, fx, fy)


def kernel(input1, input2):
    outs = []
    for b in range(B):
        tbl = input1[b].transpose(1, 2, 0).reshape(HW, C)
        fxb = input2[b, 0].reshape(HW)
        fyb = input2[b, 1].reshape(HW)
        o = _warp_image(tbl, fxb, fyb)
        outs.append(o.reshape(H, W, C).transpose(2, 0, 1))
    return jnp.stack(outs)
